# baseline (speedup 1.0000x reference)
import functools

import jax
import jax.numpy as jnp
import numpy as np
from jax import lax
from jax.experimental import pallas as pl
from jax.experimental.pallas import tpu as pltpu
from jax.experimental.pallas import tpu_sc as plsc

_INPUT_LENGTH = 65536
_DROPOUT_P = 0.001
_HOLE_LOC = 50
_HOLE_SCALE = 10
_MIN_HOLE = 1
_ROWS = 256


def _dropout_mask() -> np.ndarray:
    rng = np.random.default_rng(0)
    mask = np.zeros(_INPUT_LENGTH, dtype=bool)
    last_end = 0
    while True:
        new_gap_offset = int(rng.geometric(_DROPOUT_P)) - 1
        if new_gap_offset == 0:
            new_gap_offset = 1
        gap_start = last_end + new_gap_offset
        if gap_start >= _INPUT_LENGTH - 1:
            break
        gap_length = int(rng.normal(_HOLE_LOC, _HOLE_SCALE))
        if gap_length < _MIN_HOLE:
            gap_length = _MIN_HOLE
        gap_end = min(gap_start + gap_length, _INPUT_LENGTH)
        last_end = gap_end
        mask[gap_start:gap_end] = True
        if gap_end >= _INPUT_LENGTH:
            break
    return mask


_MASK = _dropout_mask()


def _masked_runs(mask: np.ndarray):
    runs = []
    d = np.diff(mask.astype(np.int8))
    starts = list(np.nonzero(d == 1)[0] + 1)
    ends = list(np.nonzero(d == -1)[0] + 1)
    if mask[0]:
        starts = [0] + starts
    if mask[-1]:
        ends = ends + [len(mask)]
    for s, e in zip(starts, ends):
        runs.append((int(s), int(e - s)))
    return runs


_RUNS = _masked_runs(_MASK)

_fps = []
for _a, _w in _RUNS:
    _lo = ((_a // 16) * 16) // 128 * 128
    _hi = -(-(-(-(_a + _w) // 16) * 16) // 128) * 128
    _fps.append((_lo, _hi))
_fps.sort()
_dirty = []
for _lo, _hi in _fps:
    if _dirty and _lo <= _dirty[-1][1]:
        _dirty[-1] = (_dirty[-1][0], max(_dirty[-1][1], _hi))
    else:
        _dirty.append((_lo, _hi))

_FUSE_MIN = 512
_clean_raw = []
_pos = 0
for _lo, _hi in _dirty:
    if _lo > _pos:
        _clean_raw.append((_pos, _lo - _pos))
    _pos = _hi
if _pos < _INPUT_LENGTH:
    _clean_raw.append((_pos, _INPUT_LENGTH - _pos))

_CLEAN = [(a, w) for a, w in _clean_raw if w >= _FUSE_MIN]
_DIRTY = []
_pos = 0
for _a, _w in _CLEAN:
    if _a > _pos:
        _DIRTY.append((_pos, _a - _pos))
    _pos = _a + _w
if _pos < _INPUT_LENGTH:
    _DIRTY.append((_pos, _INPUT_LENGTH - _pos))

_CLEAN_CAP = 2560
_tmp = []
for _a, _w in _CLEAN:
    while _w > _CLEAN_CAP:
        _tmp.append((_a, _CLEAN_CAP))
        _a += _CLEAN_CAP
        _w -= _CLEAN_CAP
    _tmp.append((_a, _w))
_CLEAN = _tmp
_MAX_C = max(w for _, w in _CLEAN)
_MAX_D = max(w for _, w in _DIRTY)

_rmw_masks = []
_DIRTY_OPS = []
for _A, _W in _DIRTY:
    zgroups, rgroups = [], []
    for g in range(_A, _A + _W, 16):
        seg = _MASK[g : g + 16]
        if seg.all():
            zgroups.append(g - _A)
        elif seg.any():
            rgroups.append((g - _A, len(_rmw_masks)))
            _rmw_masks.append((~seg).astype(np.float32))
    _DIRTY_OPS.append((zgroups, rgroups))

_N_RMW = len(_rmw_masks)
_RMW_TABLE_NP = (
    np.concatenate(_rmw_masks) if _N_RMW else np.zeros(16, np.float32)
)

_NC, _NS = 2, 16
_NW = _NC * _NS
_RPW = _ROWS // _NW
_RC = 3
_RD = 3
_N_CLEAN = len(_CLEAN)
_N_DIRTY = len(_DIRTY)

_mesh = plsc.VectorSubcoreMesh(core_axis_name="c", subcore_axis_name="s")


@functools.partial(
    pl.kernel,
    out_type=jax.ShapeDtypeStruct((_ROWS, _INPUT_LENGTH), jnp.float32),
    mesh=_mesh,
    scratch_types=(
        [pltpu.VMEM_SHARED((_NS, _RC, _RPW, _MAX_C), jnp.float32)]
        + [pltpu.VMEM((_RD, _RPW, _MAX_D), jnp.float32)]
        + [pltpu.VMEM((max(_N_RMW, 1) * 16,), jnp.float32)]
        + [pltpu.SemaphoreType.DMA]
        + [pltpu.SemaphoreType.DMA for _ in range(2 * _RC)]
        + [pltpu.SemaphoreType.DMA for _ in range(2 * _RD)]
    ),
)
def _sc_chunk_dropout(x_hbm, rmw_hbm, out_hbm, *scr):
    spbuf = scr[0]
    tbuf = scr[1]
    mtab = scr[2]
    sem_m = scr[3]
    cg = scr[4 : 4 + _RC]
    cs = scr[4 + _RC : 4 + 2 * _RC]
    dg = scr[4 + 2 * _RC : 4 + 2 * _RC + _RD]
    ds_ = scr[4 + 2 * _RC + _RD :]

    sid = lax.axis_index("s")
    wid = sid * _NC + lax.axis_index("c")
    r0 = pl.multiple_of(wid * _RPW, _RPW)

    mcp = pltpu.async_copy(rmw_hbm, mtab, sem_m)

    class _Path:
        def __init__(self, segs, ring, gsems, ssems, src, dst):
            self.segs, self.ring = segs, ring
            self.gsems, self.ssems = gsems, ssems
            self.src, self.dst = src, dst
            self.gathers = [None] * len(segs)
            self.pend = [None] * ring
            self.nxt = 0

        def prefetch(self, upto):
            while self.nxt <= min(upto, len(self.segs) - 1):
                g = self.nxt
                b = g % self.ring
                if self.pend[b] is not None:
                    self.pend[b].wait()
                    self.pend[b] = None
                A, W = self.segs[g]
                self.gathers[g] = pltpu.async_copy(
                    x_hbm.at[pl.ds(r0, _RPW), pl.ds(A, W)],
                    self.src(b, W),
                    self.gsems[b],
                )
                self.nxt += 1

        def step(self, i, compute=None):
            if i >= len(self.segs):
                return
            b = i % self.ring
            self.gathers[i].wait()
            if compute is not None:
                compute(i, b)
            A, W = self.segs[i]
            self.pend[b] = pltpu.async_copy(
                self.dst(b, W),
                out_hbm.at[pl.ds(r0, _RPW), pl.ds(A, W)],
                self.ssems[b],
            )
            self.prefetch(i + self.ring - 1)

        def drain(self):
            for b in range(self.ring):
                if self.pend[b] is not None:
                    self.pend[b].wait()

    def _cslot(b, W):
        return spbuf.at[sid, b, :, pl.ds(0, W)]

    def _dslot(b, W):
        return tbuf.at[b, :, pl.ds(0, W)]

    clean = _Path(_CLEAN, _RC, cg, cs, _cslot, _cslot)
    dirty = _Path(_DIRTY, _RD, dg, ds_, _dslot, _dslot)

    clean.prefetch(1)
    dirty.prefetch(1)
    mcp.wait()
    zv = jnp.zeros((16,), jnp.float32)

    def _fix(i, b):
        zgroups, rgroups = _DIRTY_OPS[i]
        if not (zgroups or rgroups):
            return

        def _row(r, _):
            for off in zgroups:
                tbuf[b, r, pl.ds(off, 16)] = zv
            for off, mi in rgroups:
                tbuf[b, r, pl.ds(off, 16)] = (
                    tbuf[b, r, pl.ds(off, 16)] * mtab[pl.ds(mi * 16, 16)]
                )
            return 0

        lax.fori_loop(0, _RPW, _row, 0)

    for i in range(max(_N_CLEAN, _N_DIRTY)):
        clean.step(i)
        dirty.step(i, _fix)
    clean.drain()
    dirty.drain()


@jax.jit
def kernel(x):
    return _sc_chunk_dropout(x, jnp.asarray(_RMW_TABLE_NP))


# speedup vs baseline: 1.0358x; 1.0358x over previous
import functools

import jax
import jax.numpy as jnp
import numpy as np
from jax import lax
from jax.experimental import pallas as pl
from jax.experimental.pallas import tpu as pltpu
from jax.experimental.pallas import tpu_sc as plsc

_INPUT_LENGTH = 65536
_DROPOUT_P = 0.001
_HOLE_LOC = 50
_HOLE_SCALE = 10
_MIN_HOLE = 1
_ROWS = 256


def _dropout_mask() -> np.ndarray:
    rng = np.random.default_rng(0)
    mask = np.zeros(_INPUT_LENGTH, dtype=bool)
    last_end = 0
    while True:
        new_gap_offset = int(rng.geometric(_DROPOUT_P)) - 1
        if new_gap_offset == 0:
            new_gap_offset = 1
        gap_start = last_end + new_gap_offset
        if gap_start >= _INPUT_LENGTH - 1:
            break
        gap_length = int(rng.normal(_HOLE_LOC, _HOLE_SCALE))
        if gap_length < _MIN_HOLE:
            gap_length = _MIN_HOLE
        gap_end = min(gap_start + gap_length, _INPUT_LENGTH)
        last_end = gap_end
        mask[gap_start:gap_end] = True
        if gap_end >= _INPUT_LENGTH:
            break
    return mask


_MASK = _dropout_mask()


def _masked_runs(mask: np.ndarray):
    runs = []
    d = np.diff(mask.astype(np.int8))
    starts = list(np.nonzero(d == 1)[0] + 1)
    ends = list(np.nonzero(d == -1)[0] + 1)
    if mask[0]:
        starts = [0] + starts
    if mask[-1]:
        ends = ends + [len(mask)]
    for s, e in zip(starts, ends):
        runs.append((int(s), int(e - s)))
    return runs


_RUNS = _masked_runs(_MASK)

_TARGET_W = 6144


def _chunk_bounds():
    fp = [((a // 16) * 16, -(-(a + w) // 16) * 16) for a, w in _RUNS]

    def clear(b):
        return all(not (lo < b < hi) for lo, hi in fp)

    bounds = [0]
    for k in range(1, -(-_INPUT_LENGTH // _TARGET_W)):
        b = k * _TARGET_W
        for off in (0, -128, 128, -256, 256, -384, 384):
            if clear(b + off):
                b = b + off
                break
        bounds.append(b)
    bounds.append(_INPUT_LENGTH)
    return bounds


_BOUNDS = _chunk_bounds()
_CHUNKS = [
    (_BOUNDS[i], _BOUNDS[i + 1] - _BOUNDS[i]) for i in range(len(_BOUNDS) - 1)
]
_MAX_W = max(w for _, w in _CHUNKS)

_rmw_masks = []
_CHUNK_OPS = []
for _A, _W in _CHUNKS:
    zgroups, rgroups = [], []
    seen = set()
    for _a, _w in _RUNS:
        if _a >= _A + _W or _a + _w <= _A:
            continue
        for g in range((_a // 16) * 16, -(-(_a + _w) // 16) * 16, 16):
            if g in seen:
                continue
            seen.add(g)
            if _MASK[g : g + 16].all():
                zgroups.append(g - _A)
            else:
                rgroups.append((g - _A, len(_rmw_masks)))
                _rmw_masks.append((~_MASK[g : g + 16]).astype(np.float32))
    _CHUNK_OPS.append((zgroups, rgroups))

_N_RMW = len(_rmw_masks)
_RMW_TABLE_NP = (
    np.concatenate(_rmw_masks) if _N_RMW else np.zeros(16, np.float32)
)

_NC, _NS = 2, 16
_NW = _NC * _NS
_RPW = _ROWS // _NW
_N_CHUNKS = len(_CHUNKS)

_mesh = plsc.VectorSubcoreMesh(core_axis_name="c", subcore_axis_name="s")


@functools.partial(
    pl.kernel,
    out_type=jax.ShapeDtypeStruct((_ROWS, _INPUT_LENGTH), jnp.float32),
    mesh=_mesh,
    scratch_types=[
        pltpu.VMEM((_RPW, _MAX_W), jnp.float32),
        pltpu.VMEM((_RPW, _MAX_W), jnp.float32),
        pltpu.VMEM((max(_N_RMW, 1) * 16,), jnp.float32),
        pltpu.SemaphoreType.DMA,
        pltpu.SemaphoreType.DMA,
        pltpu.SemaphoreType.DMA,
        pltpu.SemaphoreType.DMA,
        pltpu.SemaphoreType.DMA,
    ],
)
def _sc_chunk_dropout(
    x_hbm, rmw_hbm, out_hbm, buf0, buf1, mtab, sem_m, sg0, sg1, ss0, ss1
):
    wid = lax.axis_index("s") * _NC + lax.axis_index("c")
    r0 = pl.multiple_of(wid * _RPW, _RPW)

    bufs = (buf0, buf1)
    gsems = (sg0, sg1)
    ssems = (ss0, ss1)

    mcp = pltpu.async_copy(rmw_hbm, mtab, sem_m)

    def gather(c):
        A, W = _CHUNKS[c]
        b = c & 1
        return pltpu.async_copy(
            x_hbm.at[pl.ds(r0, _RPW), pl.ds(A, W)],
            bufs[b].at[:, pl.ds(0, W)],
            gsems[b],
        )

    def scatter(c):
        A, W = _CHUNKS[c]
        b = c & 1
        return pltpu.async_copy(
            bufs[b].at[:, pl.ds(0, W)],
            out_hbm.at[pl.ds(r0, _RPW), pl.ds(A, W)],
            ssems[b],
        )

    gathers = [None] * _N_CHUNKS
    pend_scatter = [None, None]
    gathers[0] = gather(0)
    mcp.wait()
    zv = jnp.zeros((16,), jnp.float32)

    for c in range(_N_CHUNKS):
        b = c & 1
        if c + 1 < _N_CHUNKS:
            b1 = (c + 1) & 1
            if pend_scatter[b1] is not None:
                pend_scatter[b1].wait()
                pend_scatter[b1] = None
            gathers[c + 1] = gather(c + 1)
        gathers[c].wait()

        zgroups, rgroups = _CHUNK_OPS[c]
        if zgroups or rgroups:
            buf = bufs[b]

            def _rowpair(rp, _):
                for rr in range(2):
                    r = rp * 2 + rr
                    for off in zgroups:
                        buf[r, pl.ds(off, 16)] = zv
                    for off, mi in rgroups:
                        buf[r, pl.ds(off, 16)] = (
                            buf[r, pl.ds(off, 16)] * mtab[pl.ds(mi * 16, 16)]
                        )
                return 0

            lax.fori_loop(0, _RPW // 2, _rowpair, 0)

        pend_scatter[b] = scatter(c)

    for b in (0, 1):
        if pend_scatter[b] is not None:
            pend_scatter[b].wait()


@jax.jit
def kernel(x):
    return _sc_chunk_dropout(x, jnp.asarray(_RMW_TABLE_NP))


# speedup vs baseline: 1.1440x; 1.1045x over previous
import functools

import jax
import jax.numpy as jnp
import numpy as np
from jax import lax
from jax.experimental import pallas as pl
from jax.experimental.pallas import tpu as pltpu
from jax.experimental.pallas import tpu_sc as plsc

_INPUT_LENGTH = 65536
_DROPOUT_P = 0.001
_HOLE_LOC = 50
_HOLE_SCALE = 10
_MIN_HOLE = 1
_ROWS = 256


def _dropout_mask() -> np.ndarray:
    rng = np.random.default_rng(0)
    mask = np.zeros(_INPUT_LENGTH, dtype=bool)
    last_end = 0
    while True:
        new_gap_offset = int(rng.geometric(_DROPOUT_P)) - 1
        if new_gap_offset == 0:
            new_gap_offset = 1
        gap_start = last_end + new_gap_offset
        if gap_start >= _INPUT_LENGTH - 1:
            break
        gap_length = int(rng.normal(_HOLE_LOC, _HOLE_SCALE))
        if gap_length < _MIN_HOLE:
            gap_length = _MIN_HOLE
        gap_end = min(gap_start + gap_length, _INPUT_LENGTH)
        last_end = gap_end
        mask[gap_start:gap_end] = True
        if gap_end >= _INPUT_LENGTH:
            break
    return mask


_MASK = _dropout_mask()


def _masked_runs(mask: np.ndarray):
    runs = []
    d = np.diff(mask.astype(np.int8))
    starts = list(np.nonzero(d == 1)[0] + 1)
    ends = list(np.nonzero(d == -1)[0] + 1)
    if mask[0]:
        starts = [0] + starts
    if mask[-1]:
        ends = ends + [len(mask)]
    for s, e in zip(starts, ends):
        runs.append((int(s), int(e - s)))
    return runs


_RUNS = _masked_runs(_MASK)

_TARGET_W = 6144


def _chunk_bounds():
    fp = [((a // 16) * 16, -(-(a + w) // 16) * 16) for a, w in _RUNS]

    def clear(b):
        return all(not (lo < b < hi) for lo, hi in fp)

    bounds = [0]
    for k in range(1, -(-_INPUT_LENGTH // _TARGET_W)):
        b = k * _TARGET_W
        for off in (0, -128, 128, -256, 256, -384, 384):
            if clear(b + off):
                b = b + off
                break
        bounds.append(b)
    bounds.append(_INPUT_LENGTH)
    return bounds


_BOUNDS = _chunk_bounds()
_CHUNKS = [
    (_BOUNDS[i], _BOUNDS[i + 1] - _BOUNDS[i]) for i in range(len(_BOUNDS) - 1)
]
_MAX_W = max(w for _, w in _CHUNKS)

_rmw_masks = []
_CHUNK_OPS = []
for _A, _W in _CHUNKS:
    zgroups, rgroups = [], []
    seen = set()
    for _a, _w in _RUNS:
        if _a >= _A + _W or _a + _w <= _A:
            continue
        for g in range((_a // 16) * 16, -(-(_a + _w) // 16) * 16, 16):
            if g in seen:
                continue
            seen.add(g)
            if _MASK[g : g + 16].all():
                zgroups.append(g - _A)
            else:
                rgroups.append((g - _A, len(_rmw_masks)))
                _rmw_masks.append((~_MASK[g : g + 16]).astype(np.float32))
    _CHUNK_OPS.append((zgroups, rgroups))

_N_RMW = len(_rmw_masks)
_RMW_TABLE_NP = (
    np.concatenate(_rmw_masks) if _N_RMW else np.zeros(16, np.float32)
)

_NC, _NS = 2, 16
_NW = _NC * _NS
_TC_ROWS = 128
_SC_ROWS = _ROWS - _TC_ROWS
_RPW = _SC_ROWS // _NW
_N_CHUNKS = len(_CHUNKS)
_KEEP_NP = (~_MASK).astype(np.float32)[None, :]
_TC_BLOCK_W = 8192

_mesh = plsc.VectorSubcoreMesh(core_axis_name="c", subcore_axis_name="s")


@functools.partial(
    pl.kernel,
    out_type=jax.ShapeDtypeStruct((_SC_ROWS, _INPUT_LENGTH), jnp.float32),
    mesh=_mesh,
    scratch_types=[
        pltpu.VMEM((_RPW, _MAX_W), jnp.float32),
        pltpu.VMEM((_RPW, _MAX_W), jnp.float32),
        pltpu.VMEM((max(_N_RMW, 1) * 16,), jnp.float32),
        pltpu.SemaphoreType.DMA,
        pltpu.SemaphoreType.DMA,
        pltpu.SemaphoreType.DMA,
        pltpu.SemaphoreType.DMA,
        pltpu.SemaphoreType.DMA,
    ],
)
def _sc_chunk_dropout(
    x_hbm, rmw_hbm, out_hbm, buf0, buf1, mtab, sem_m, sg0, sg1, ss0, ss1
):
    wid = lax.axis_index("s") * _NC + lax.axis_index("c")
    r0 = pl.multiple_of(wid * _RPW, _RPW)
    rx = pl.multiple_of(_TC_ROWS + wid * _RPW, _RPW)

    bufs = (buf0, buf1)
    gsems = (sg0, sg1)
    ssems = (ss0, ss1)

    mcp = pltpu.async_copy(rmw_hbm, mtab, sem_m)

    def gather(c):
        A, W = _CHUNKS[c]
        b = c & 1
        return pltpu.async_copy(
            x_hbm.at[pl.ds(rx, _RPW), pl.ds(A, W)],
            bufs[b].at[:, pl.ds(0, W)],
            gsems[b],
        )

    def scatter(c):
        A, W = _CHUNKS[c]
        b = c & 1
        return pltpu.async_copy(
            bufs[b].at[:, pl.ds(0, W)],
            out_hbm.at[pl.ds(r0, _RPW), pl.ds(A, W)],
            ssems[b],
        )

    gathers = [None] * _N_CHUNKS
    pend_scatter = [None, None]
    gathers[0] = gather(0)
    mcp.wait()
    zv = jnp.zeros((16,), jnp.float32)

    for c in range(_N_CHUNKS):
        b = c & 1
        if c + 1 < _N_CHUNKS:
            b1 = (c + 1) & 1
            if pend_scatter[b1] is not None:
                pend_scatter[b1].wait()
                pend_scatter[b1] = None
            gathers[c + 1] = gather(c + 1)
        gathers[c].wait()

        zgroups, rgroups = _CHUNK_OPS[c]
        if zgroups or rgroups:
            buf = bufs[b]

            def _rowpair(rp, _):
                for rr in range(2):
                    r = rp * 2 + rr
                    for off in zgroups:
                        buf[r, pl.ds(off, 16)] = zv
                    for off, mi in rgroups:
                        buf[r, pl.ds(off, 16)] = (
                            buf[r, pl.ds(off, 16)] * mtab[pl.ds(mi * 16, 16)]
                        )
                return 0

            lax.fori_loop(0, _RPW // 2, _rowpair, 0)

        pend_scatter[b] = scatter(c)

    for b in (0, 1):
        if pend_scatter[b] is not None:
            pend_scatter[b].wait()


def _tc_mask_mul(x_ref, m_ref, o_ref):
    o_ref[...] = x_ref[...] * m_ref[...]


def _tc_part(x, m):
    return pl.pallas_call(
        _tc_mask_mul,
        grid=(_INPUT_LENGTH // _TC_BLOCK_W,),
        in_specs=[
            pl.BlockSpec((_TC_ROWS, _TC_BLOCK_W), lambda j: (0, j)),
            pl.BlockSpec((1, _TC_BLOCK_W), lambda j: (0, j)),
        ],
        out_specs=pl.BlockSpec((_TC_ROWS, _TC_BLOCK_W), lambda j: (0, j)),
        out_shape=jax.ShapeDtypeStruct((_TC_ROWS, _INPUT_LENGTH), jnp.float32),
    )(x, m)


@jax.jit
def kernel(x):
    sc = _sc_chunk_dropout(x, jnp.asarray(_RMW_TABLE_NP))
    tc = _tc_part(x, jnp.asarray(_KEEP_NP))
    return tc, sc
